# Initial kernel scaffold; baseline (speedup 1.0000x reference)
import jax
import jax.numpy as jnp
from jax.experimental import pallas as pl


def kernel(x, weight):
    raise NotImplementedError("write your pallas kernel here")


# speedup vs baseline: 4.2474x; 4.2474x over previous
import functools

import jax
import jax.numpy as jnp
from jax import lax
from jax.experimental import pallas as pl
from jax.experimental.pallas import tpu as pltpu
from jax.experimental.pallas import tpu_sc as plsc

_VOCAB = 1000000
_D = 32
_B = 16384
_H = 20
_L = 16
_NC, _NS = 2, 16
_NW = _NC * _NS
_BAGS_PER_W = _B // _NW
_CHUNK = 64
_NCHUNK = _BAGS_PER_W // _CHUNK
_ROWS_PER_CHUNK = _CHUNK * _H
_IDX_COLS = 128
_IDX_ROWS_PER_CHUNK = _ROWS_PER_CHUNK // _IDX_COLS
_IDX_ROWS_PER_W = _BAGS_PER_W * _H // _IDX_COLS


def _quant_mag(z):
    return jnp.where(z > 5.0, 6.0,
           jnp.where(z > 3.5, 4.0,
           jnp.where(z > 2.5, 3.0,
           jnp.where(z > 1.75, 2.0,
           jnp.where(z > 1.25, 1.5,
           jnp.where(z > 0.75, 1.0,
           jnp.where(z > 0.25, 0.5, 0.0)))))))


def _idx_body(x_ref, out_ref):
    xf = x_ref[...].astype(jnp.float32)
    col = lax.broadcasted_iota(jnp.int32, xf.shape, 1)
    is0 = col < _L
    ax = jnp.abs(xf)
    m0 = jnp.max(jnp.where(is0, ax, 0.0), axis=1, keepdims=True)
    m1 = jnp.max(jnp.where(is0, 0.0, ax), axis=1, keepdims=True)
    amax = jnp.where(is0, m0, m1)
    scale = jnp.where(amax > 0, amax / 6.0, 1.0)
    y = xf / scale
    z = jnp.abs(y)
    qm = _quant_mag(z)
    dq = jnp.where(y < 0, -qm, qm) * scale
    r = lax.round(dq, lax.RoundingMethod.TO_NEAREST_EVEN)
    out_ref[...] = jnp.clip(r, 0.0, float(_VOCAB - 1)).astype(jnp.int32)


_idx_call = pl.pallas_call(
    _idx_body,
    out_shape=jax.ShapeDtypeStruct((_B, _H), jnp.int32),
    grid=(16,),
    in_specs=[pl.BlockSpec((_B // 16, _H), lambda i: (i, 0))],
    out_specs=pl.BlockSpec((_B // 16, _H), lambda i: (i, 0)),
)


_GATHER_DNUMS = lax.GatherDimensionNumbers(
    offset_dims=(), collapsed_slice_dims=(0,), start_index_map=(0,))


def _shuffle(v, perm):
    return lax.gather(v, perm[:, None], _GATHER_DNUMS, (1,),
                      mode=lax.GatherScatterMode.PROMISE_IN_BOUNDS)


def _lanemax(v):
    for s in (8, 4, 2, 1):
        perm = lax.iota(jnp.int32, _L) ^ s
        v = jnp.maximum(v, _shuffle(v, perm))
    return v


def _qdq16(w):
    a = _lanemax(jnp.abs(w))
    scale = jnp.where(a > 0.0, a / 6.0, 1.0)
    y = w / scale
    qm = _quant_mag(jnp.abs(y))
    q = jnp.where(y < 0.0, -qm, qm)
    return q * scale


def _sc_body(idx_hbm, table_hbm, out_hbm, idx_v, rows_v, out_v, sem):
    wid = lax.axis_index("s") * _NC + lax.axis_index("c")
    pltpu.sync_copy(idx_hbm.at[pl.ds(wid * _IDX_ROWS_PER_W, _IDX_ROWS_PER_W)],
                    idx_v)

    def chunk_body(c, carry):
        copies = []
        for j in range(_IDX_ROWS_PER_CHUNK):
            copies.append(pltpu.async_copy(
                table_hbm.at[idx_v.at[c * _IDX_ROWS_PER_CHUNK + j]],
                rows_v.at[pl.ds(j * _IDX_COLS, _IDX_COLS)], sem))
        for cp in copies:
            cp.wait()

        def bag_body(b, carry2):
            r0 = b * _H
            acc0 = jnp.zeros((_L,), jnp.float32)
            acc1 = jnp.zeros((_L,), jnp.float32)
            for k in range(_H):
                acc0 = acc0 + _qdq16(rows_v[r0 + k, pl.ds(0, _L)])
                acc1 = acc1 + _qdq16(rows_v[r0 + k, pl.ds(_L, _L)])
            out_v[b, pl.ds(0, _L)] = acc0 / float(_H)
            out_v[b, pl.ds(_L, _L)] = acc1 / float(_H)
            return carry2

        lax.fori_loop(0, _CHUNK, bag_body, 0)
        bag0 = wid * _BAGS_PER_W + c * _CHUNK
        pltpu.sync_copy(out_v, out_hbm.at[pl.ds(bag0, _CHUNK)])
        return carry

    lax.fori_loop(0, _NCHUNK, chunk_body, 0)


@functools.cache
def _sc_call():
    return pl.kernel(
        _sc_body,
        out_type=jax.ShapeDtypeStruct((_B, _D), jnp.float32),
        mesh=plsc.VectorSubcoreMesh(core_axis_name="c", subcore_axis_name="s"),
        compiler_params=pltpu.CompilerParams(use_tc_tiling_on_sc=False),
        scratch_types=[
            pltpu.VMEM((_IDX_ROWS_PER_W, _IDX_COLS), jnp.int32),
            pltpu.VMEM((_ROWS_PER_CHUNK, _D), jnp.float32),
            pltpu.VMEM((_CHUNK, _D), jnp.float32),
            pltpu.SemaphoreType.DMA,
        ],
    )


def kernel(x, weight):
    x = x.astype(jnp.int32)
    idx = _idx_call(x)
    idx2d = idx.reshape(_B * _H // _IDX_COLS, _IDX_COLS)
    return _sc_call()(idx2d, weight)
